# baseline (speedup 1.0000x reference)
import functools

import jax
import jax.numpy as jnp
from jax import lax
from jax.experimental import pallas as pl
from jax.experimental.pallas import tpu as pltpu
from jax.experimental.pallas import tpu_sc as plsc

VOCAB = 1000000
EMBED_DIM = 64
BATCH = 16384

NC = 2
NS = 16
NW = NC * NS

CHUNK = 128
N_CHUNK_ROWS = BATCH // CHUNK
ROWS_PER_W = N_CHUNK_ROWS // NW


def _gather_body(target_hbm, other_hbm, table_hbm, out_t_hbm, out_o_hbm,
                 idx_t, idx_o, rows_t, rows_o, sem):
    wid = lax.axis_index("s") * NC + lax.axis_index("c")
    base = wid * ROWS_PER_W

    pltpu.sync_copy(target_hbm.at[pl.ds(base, ROWS_PER_W)], idx_t)
    pltpu.sync_copy(other_hbm.at[pl.ds(base, ROWS_PER_W)], idx_o)

    copies = []
    for j in range(ROWS_PER_W):
        copies.append(
            pltpu.async_copy(table_hbm.at[idx_t.at[j]], rows_t.at[j], sem))
        copies.append(
            pltpu.async_copy(table_hbm.at[idx_o.at[j]], rows_o.at[j], sem))
    for c in copies:
        c.wait()

    pltpu.sync_copy(rows_t, out_t_hbm.at[pl.ds(base, ROWS_PER_W)])
    pltpu.sync_copy(rows_o, out_o_hbm.at[pl.ds(base, ROWS_PER_W)])


@jax.jit
def _skipgram_gather(target2d, other2d, table):
    mesh = plsc.VectorSubcoreMesh(core_axis_name="c", subcore_axis_name="s")
    out_sds = jax.ShapeDtypeStruct((N_CHUNK_ROWS, CHUNK, EMBED_DIM),
                                   jnp.float32)
    run = pl.kernel(
        _gather_body,
        out_type=(out_sds, out_sds),
        mesh=mesh,
        compiler_params=pltpu.CompilerParams(use_tc_tiling_on_sc=False),
        scratch_types=[
            pltpu.VMEM((ROWS_PER_W, CHUNK), jnp.int32),
            pltpu.VMEM((ROWS_PER_W, CHUNK), jnp.int32),
            pltpu.VMEM((ROWS_PER_W, CHUNK, EMBED_DIM), jnp.float32),
            pltpu.VMEM((ROWS_PER_W, CHUNK, EMBED_DIM), jnp.float32),
            pltpu.SemaphoreType.DMA,
        ],
    )
    return run(target2d, other2d, table)


def kernel(target, other, table):
    target2d = target.astype(jnp.int32).reshape(N_CHUNK_ROWS, CHUNK)
    other2d = other.astype(jnp.int32).reshape(N_CHUNK_ROWS, CHUNK)
    out_t, out_o = _skipgram_gather(target2d, other2d, table)
    return (out_t.reshape(BATCH, EMBED_DIM), out_o.reshape(BATCH, EMBED_DIM))
